# Initial kernel scaffold; baseline (speedup 1.0000x reference)
import jax
import jax.numpy as jnp
from jax.experimental import pallas as pl


def kernel(x, edge_index, batch, W1, b1, g1, bt1, W2, b2, g2, bt2, fcW1, fcb1, g3, bt3, fcW2, fcb2, fcW3, fcb3):
    raise NotImplementedError("write your pallas kernel here")


# speedup vs baseline: 11.7155x; 11.7155x over previous
import functools

import jax
import jax.numpy as jnp
from jax import lax
from jax.experimental import pallas as pl
from jax.experimental.pallas import tpu as pltpu
from jax.experimental.pallas import tpu_sc as plsc

NC = 2
NS = 16
K = 128
BN_EPS = 1e-5


def _cdiv(a, b):
    return (a + b - 1) // b


def _pad_rows(n):
    return _cdiv(n, NS * 8) * (NS * 8)


def _sc_mesh():
    return plsc.VectorSubcoreMesh(core_axis_name="c", subcore_axis_name="s")


def _zero_acc(acc, zb, s, rows_per_tile, zrows, w):
    zvec = jnp.zeros((16,), jnp.float32)

    def zrow(i, _):
        for t in range(w // 16):
            zb[i, pl.ds(t * 16, 16)] = zvec
        return 0

    lax.fori_loop(0, zrows, zrow, 0)
    base = s * rows_per_tile

    def zcp(i, _):
        pltpu.sync_copy(zb, acc.at[pl.ds(base + i * zrows, zrows)])
        return 0

    lax.fori_loop(0, rows_per_tile // zrows, zcp, 0)


def _make_deg_kernel(n, e):
    nchunks = e // K
    chunks_per_core = nchunks // NC
    per_tile = _cdiv(chunks_per_core, NS)
    np_ = _pad_rows(n)
    rows_per_tile = np_ // NS
    zrows = rows_per_tile // 8

    @functools.partial(
        pl.kernel,
        out_type=jax.ShapeDtypeStruct((NC, np_, 16), jnp.float32),
        mesh=_sc_mesh(),
        compiler_params=pltpu.CompilerParams(use_tc_tiling_on_sc=False),
        scratch_types=[
            pltpu.VMEM_SHARED((np_, 16), jnp.float32),
            pltpu.VMEM((K,), jnp.int32),
            pltpu.VMEM((K, 16), jnp.float32),
            pltpu.VMEM((zrows, 16), jnp.float32),
        ],
    )
    def deg_kernel(dst_hbm, out_hbm, acc, idxb, oneb, zb):
        c = lax.axis_index("c")
        s = lax.axis_index("s")
        onehot0 = jnp.where(lax.iota(jnp.int32, 16) == 0, 1.0, 0.0)

        def orow(i, _):
            oneb[i, :] = onehot0
            return 0

        lax.fori_loop(0, K, orow, 0)
        _zero_acc(acc, zb, s, rows_per_tile, zrows, 16)
        plsc.subcore_barrier()

        cbase = c * chunks_per_core

        def step(i, _):
            jj = s + i * NS

            @pl.when(jj < chunks_per_core)
            def _():
                j = cbase + jj
                pltpu.sync_copy(dst_hbm.at[pl.ds(j * K, K)], idxb)
                pltpu.sync_copy(oneb, acc.at[idxb], add=True)

            return 0

        lax.fori_loop(0, per_tile, step, 0)
        plsc.subcore_barrier()
        base = s * rows_per_tile
        pltpu.sync_copy(acc.at[pl.ds(base, rows_per_tile)],
                        out_hbm.at[c, pl.ds(base, rows_per_tile)])

    return deg_kernel


def _make_agg_kernel(n, e, w, split_edges):
    nchunks = e // K
    chunks_per_core = nchunks // NC if split_edges else nchunks
    per_tile = _cdiv(chunks_per_core, NS)
    np_ = _pad_rows(n)
    rows_per_tile = np_ // NS
    zrows = rows_per_tile // 8

    @functools.partial(
        pl.kernel,
        out_type=jax.ShapeDtypeStruct((NC, np_, w), jnp.float32),
        mesh=_sc_mesh(),
        compiler_params=pltpu.CompilerParams(use_tc_tiling_on_sc=False),
        scratch_types=[
            pltpu.VMEM_SHARED((np_, w), jnp.float32),
            pltpu.VMEM((K,), jnp.int32),
            pltpu.VMEM((K,), jnp.int32),
            pltpu.VMEM((K, w), jnp.float32),
            pltpu.VMEM((zrows, w), jnp.float32),
            pltpu.SemaphoreType.DMA,
        ],
    )
    def agg_kernel(src_hbm, dst_hbm, tab_hbm, out_hbm,
                   acc, srcb, dstb, rowb, zb, sem):
        c = lax.axis_index("c")
        s = lax.axis_index("s")
        _zero_acc(acc, zb, s, rows_per_tile, zrows, w)
        plsc.subcore_barrier()

        cbase = c * chunks_per_core if split_edges else 0

        def step(i, _):
            jj = s + i * NS

            @pl.when(jj < chunks_per_core)
            def _():
                j = cbase + jj
                pltpu.sync_copy(src_hbm.at[pl.ds(j * K, K)], srcb)
                pltpu.sync_copy(dst_hbm.at[pl.ds(j * K, K)], dstb)
                if not split_edges:
                    for t in range(K // 16):
                        srcb[pl.ds(t * 16, 16)] = (
                            srcb[pl.ds(t * 16, 16)] + c * n)
                pltpu.async_copy(tab_hbm.at[srcb], rowb, sem).wait()
                pltpu.sync_copy(rowb, acc.at[dstb], add=True)

            return 0

        lax.fori_loop(0, per_tile, step, 0)
        plsc.subcore_barrier()
        base = s * rows_per_tile
        pltpu.sync_copy(acc.at[pl.ds(base, rows_per_tile)],
                        out_hbm.at[c, pl.ds(base, rows_per_tile)])

    return agg_kernel


def _tc_dinv_t1(degp, xpad, n, r):
    nb = n // r

    def body(degp_ref, x_ref, dinv_ref, t1_ref):
        dp = degp_ref[...]
        deg = dp[0, :, 0:1] + dp[1, :, 0:1] + 1.0
        dinv = lax.rsqrt(deg)
        dinv16 = jnp.broadcast_to(dinv, (r, 16))
        dinv_ref[...] = dinv16
        t1_ref[...] = dinv16 * x_ref[...]

    return pl.pallas_call(
        body,
        grid=(nb,),
        in_specs=[
            pl.BlockSpec((NC, r, 16), lambda i: (0, i, 0)),
            pl.BlockSpec((r, 16), lambda i: (i, 0)),
        ],
        out_specs=[
            pl.BlockSpec((r, 16), lambda i: (i, 0)),
            pl.BlockSpec((r, 16), lambda i: (i, 0)),
        ],
        out_shape=[
            jax.ShapeDtypeStruct((n, 16), jnp.float32),
            jax.ShapeDtypeStruct((n, 16), jnp.float32),
        ],
    )(degp, xpad)


def _tc_layer1(p1, t1, dinv16, W1p, b1r, n, r):
    nb = n // r

    def body(p1_ref, t1_ref, dinv_ref, w_ref, b_ref, h_ref, s_ref, q_ref):
        i = pl.program_id(0)
        p = p1_ref[...]
        accv = p[0] + p[1] + t1_ref[...]
        agg = dinv_ref[...] * accv
        h = jnp.dot(agg, w_ref[...], preferred_element_type=jnp.float32)
        h = jnp.maximum(h + b_ref[...], 0.0)
        h_ref[...] = h

        @pl.when(i == 0)
        def _():
            s_ref[...] = jnp.zeros_like(s_ref)
            q_ref[...] = jnp.zeros_like(q_ref)

        hr = h.reshape(r // 8, 8, 128)
        s_ref[...] += jnp.sum(hr, axis=0)
        q_ref[...] += jnp.sum(hr * hr, axis=0)

    return pl.pallas_call(
        body,
        grid=(nb,),
        in_specs=[
            pl.BlockSpec((NC, r, 16), lambda i: (0, i, 0)),
            pl.BlockSpec((r, 16), lambda i: (i, 0)),
            pl.BlockSpec((r, 16), lambda i: (i, 0)),
            pl.BlockSpec((16, 128), lambda i: (0, 0)),
            pl.BlockSpec((1, 128), lambda i: (0, 0)),
        ],
        out_specs=[
            pl.BlockSpec((r, 128), lambda i: (i, 0)),
            pl.BlockSpec((8, 128), lambda i: (0, 0)),
            pl.BlockSpec((8, 128), lambda i: (0, 0)),
        ],
        out_shape=[
            jax.ShapeDtypeStruct((n, 128), jnp.float32),
            jax.ShapeDtypeStruct((8, 128), jnp.float32),
            jax.ShapeDtypeStruct((8, 128), jnp.float32),
        ],
    )(p1, t1, dinv16, W1p, b1r)


def _tc_bn1_w2(h1, s1, q1, g1r, bt1r, W2, dinv16, n, r):
    nb = n // r
    fn = float(n)

    def body(h_ref, s_ref, q_ref, g_ref, b_ref, w_ref, dinv_ref,
             ta_ref, tb_ref):
        mean = jnp.sum(s_ref[...], axis=0, keepdims=True) / fn
        var = jnp.sum(q_ref[...], axis=0, keepdims=True) / fn - mean * mean
        istd = lax.rsqrt(var + BN_EPS)
        hb = (h_ref[...] - mean) * istd * g_ref[...] + b_ref[...]
        hw = jnp.dot(hb, w_ref[...], preferred_element_type=jnp.float32)
        t2 = dinv_ref[...][:, 0:1] * hw
        ta_ref[...] = t2[:, :32]
        tb_ref[...] = t2[:, 32:]

    return pl.pallas_call(
        body,
        grid=(nb,),
        in_specs=[
            pl.BlockSpec((r, 128), lambda i: (i, 0)),
            pl.BlockSpec((8, 128), lambda i: (0, 0)),
            pl.BlockSpec((8, 128), lambda i: (0, 0)),
            pl.BlockSpec((1, 128), lambda i: (0, 0)),
            pl.BlockSpec((1, 128), lambda i: (0, 0)),
            pl.BlockSpec((128, 64), lambda i: (0, 0)),
            pl.BlockSpec((r, 16), lambda i: (i, 0)),
        ],
        out_specs=[
            pl.BlockSpec((r, 32), lambda i: (i, 0)),
            pl.BlockSpec((r, 32), lambda i: (i, 0)),
        ],
        out_shape=[
            jax.ShapeDtypeStruct((n, 32), jnp.float32),
            jax.ShapeDtypeStruct((n, 32), jnp.float32),
        ],
    )(h1, s1, q1, g1r, bt1r, W2, dinv16)


def _tc_layer2(p2, t2s, dinv16, b2r, n, r):
    nb = n // r

    def body(p2_ref, t2_ref, dinv_ref, b_ref, h_ref, s_ref, q_ref):
        i = pl.program_id(1)
        accv = p2_ref[...][0] + t2_ref[...]
        h = dinv_ref[...][:, 0:1] * accv + b_ref[...][0]
        h = jnp.maximum(h, 0.0)
        h_ref[...] = h[None]

        @pl.when(i == 0)
        def _():
            s_ref[...] = jnp.zeros_like(s_ref)
            q_ref[...] = jnp.zeros_like(q_ref)

        hr = h.reshape(r // 8, 8, 32)
        s_ref[...] += jnp.sum(hr, axis=0)[None]
        q_ref[...] += jnp.sum(hr * hr, axis=0)[None]

    return pl.pallas_call(
        body,
        grid=(NC, nb),
        in_specs=[
            pl.BlockSpec((1, r, 32), lambda c, i: (c, i, 0)),
            pl.BlockSpec((r, 32), lambda c, i: (c * nb + i, 0)),
            pl.BlockSpec((r, 16), lambda c, i: (i, 0)),
            pl.BlockSpec((1, 1, 32), lambda c, i: (c, 0, 0)),
        ],
        out_specs=[
            pl.BlockSpec((1, r, 32), lambda c, i: (c, i, 0)),
            pl.BlockSpec((1, 8, 32), lambda c, i: (c, 0, 0)),
            pl.BlockSpec((1, 8, 32), lambda c, i: (c, 0, 0)),
        ],
        out_shape=[
            jax.ShapeDtypeStruct((NC, n, 32), jnp.float32),
            jax.ShapeDtypeStruct((NC, 8, 32), jnp.float32),
            jax.ShapeDtypeStruct((NC, 8, 32), jnp.float32),
        ],
    )(p2, t2s, dinv16, b2r)


def _tc_bn2_pool(h2, s2, q2, g2r, bt2r, batch3, n, r, g):
    nb = n // r
    fn = float(n)

    def body(h_ref, s_ref, q_ref, g_ref, b_ref, bat_ref, p_ref):
        i = pl.program_id(1)
        mean = jnp.sum(s_ref[...][0], axis=0, keepdims=True) / fn
        var = jnp.sum(q_ref[...][0], axis=0, keepdims=True) / fn - mean * mean
        istd = lax.rsqrt(var + BN_EPS)
        hb = (h_ref[...][0] - mean) * istd * g_ref[...][0] + b_ref[...][0]
        bat = bat_ref[...][0]
        oh = (lax.broadcasted_iota(jnp.int32, (g, r), 0) == bat
              ).astype(jnp.float32)

        @pl.when(i == 0)
        def _():
            p_ref[...] = jnp.zeros_like(p_ref)

        p_ref[...] += jnp.dot(oh, hb, preferred_element_type=jnp.float32)[None]

    return pl.pallas_call(
        body,
        grid=(NC, nb),
        in_specs=[
            pl.BlockSpec((1, r, 32), lambda c, i: (c, i, 0)),
            pl.BlockSpec((1, 8, 32), lambda c, i: (c, 0, 0)),
            pl.BlockSpec((1, 8, 32), lambda c, i: (c, 0, 0)),
            pl.BlockSpec((1, 1, 32), lambda c, i: (c, 0, 0)),
            pl.BlockSpec((1, 1, 32), lambda c, i: (c, 0, 0)),
            pl.BlockSpec((1, 1, r), lambda c, i: (i, 0, 0)),
        ],
        out_specs=pl.BlockSpec((1, g, 32), lambda c, i: (c, 0, 0)),
        out_shape=jax.ShapeDtypeStruct((NC, g, 32), jnp.float32),
    )(h2, s2, q2, g2r, bt2r, batch3)


def _tc_head(pooled, fcW1, fcb1r, g3r, bt3r, fcW2, fcb2r, fcW3p, fcb3p, g):
    fg = float(g)

    def body(p_ref, w1_ref, b1_ref, g3_ref, bt3_ref, w2_ref, b2_ref,
             w3_ref, b3_ref, o_ref):
        p = jnp.dot(p_ref[...], w1_ref[...],
                    preferred_element_type=jnp.float32)
        p = jnp.maximum(p + b1_ref[...], 0.0)
        mean = jnp.sum(p, axis=0, keepdims=True) / fg
        var = jnp.sum(p * p, axis=0, keepdims=True) / fg - mean * mean
        p = (p - mean) * lax.rsqrt(var + BN_EPS) * g3_ref[...] + bt3_ref[...]
        p = jnp.dot(p, w2_ref[...], preferred_element_type=jnp.float32)
        p = jnp.maximum(p + b2_ref[...], 0.0)
        z = jnp.dot(p, w3_ref[...],
                    preferred_element_type=jnp.float32) + b3_ref[...]
        m = jnp.max(z, axis=1, keepdims=True)
        lse = m + jnp.log(jnp.sum(jnp.exp(z - m), axis=1, keepdims=True))
        o_ref[...] = z - lse

    return pl.pallas_call(
        body,
        in_specs=[
            pl.BlockSpec((g, 64), lambda: (0, 0)),
            pl.BlockSpec((64, 64), lambda: (0, 0)),
            pl.BlockSpec((1, 64), lambda: (0, 0)),
            pl.BlockSpec((1, 64), lambda: (0, 0)),
            pl.BlockSpec((1, 64), lambda: (0, 0)),
            pl.BlockSpec((64, 64), lambda: (0, 0)),
            pl.BlockSpec((1, 64), lambda: (0, 0)),
            pl.BlockSpec((64, 128), lambda: (0, 0)),
            pl.BlockSpec((1, 128), lambda: (0, 0)),
        ],
        out_specs=pl.BlockSpec((g, 128), lambda: (0, 0)),
        out_shape=jax.ShapeDtypeStruct((g, 128), jnp.float32),
    )(pooled, fcW1, fcb1r, g3r, bt3r, fcW2, fcb2r, fcW3p, fcb3p)


def kernel(x, edge_index, batch, W1, b1, g1, bt1, W2, b2, g2, bt2,
           fcW1, fcb1, g3, bt3, fcW2, fcb2, fcW3, fcb3):
    n, f0 = x.shape
    e = edge_index.shape[1]
    g = 64
    r = 400

    ei = edge_index.astype(jnp.int32)
    src = ei[0]
    dst = ei[1]

    degp = _make_deg_kernel(n, e)(dst)

    xpad = jnp.pad(x, ((0, 0), (0, 16 - f0)))
    dinv16, t1 = _tc_dinv_t1(degp, xpad, n, r)

    p1 = _make_agg_kernel(n, e, 16, split_edges=True)(src, dst, t1)

    W1p = jnp.pad(W1, ((0, 16 - f0), (0, 0)))
    h1, s1, q1 = _tc_layer1(p1, t1, dinv16, W1p, b1.reshape(1, -1), n, r)

    t2a, t2b = _tc_bn1_w2(h1, s1, q1, g1.reshape(1, -1), bt1.reshape(1, -1),
                          W2, dinv16, n, r)
    t2s = jnp.concatenate([t2a, t2b], axis=0)

    p2 = _make_agg_kernel(n, e, 32, split_edges=False)(src, dst, t2s)

    h2, s2, q2 = _tc_layer2(p2, t2s, dinv16, b2.reshape(NC, 1, 32), n, r)

    batch3 = batch.astype(jnp.int32).reshape(n // r, 1, r)
    pooled2 = _tc_bn2_pool(h2, s2, q2, g2.reshape(NC, 1, 32),
                           bt2.reshape(NC, 1, 32), batch3, n, r, g)
    pooled = jnp.concatenate([pooled2[0], pooled2[1]], axis=1)

    nclass = fcW3.shape[1]
    fcW3p = jnp.pad(fcW3, ((0, 0), (0, 128 - nclass)))
    fcb3p = jnp.pad(fcb3.reshape(1, -1), ((0, 0), (0, 128 - nclass)),
                    constant_values=-1e30)
    out = _tc_head(pooled, fcW1, fcb1.reshape(1, -1), g3.reshape(1, -1),
                   bt3.reshape(1, -1), fcW2, fcb2.reshape(1, -1),
                   fcW3p, fcb3p, g)
    return out[:, :nclass]
